# baseline (speedup 1.0000x reference)
import jax
import jax.numpy as jnp
from jax.experimental import pallas as pl
from jax.experimental.pallas import tpu as pltpu

_B = 32
_M = 30
_SHAPES = [(64, 64, 64), (128, 32, 32), (256, 16, 16)]
_DS = [c * h * w for (c, h, w) in _SHAPES]
_NCHUNK = 8


def _partial_dist(a, b, d):
    cross = jax.lax.dot_general(
        a, b, (((1,), (1,)), ((), ())), preferred_element_type=jnp.float32
    )
    a2 = jnp.sum(a * a, axis=1)
    b2 = jnp.sum(b * b, axis=1)
    return (a2[:, None] + b2[None, :] - 2.0 * cross) * (1.0 / d)


def _dist1_kernel(f1, m1, out, acc):
    g = pl.program_id(0)

    @pl.when(g == 0)
    def _init():
        acc[:] = jnp.zeros_like(acc)

    acc[:] += _partial_dist(f1[:], m1[:], _DS[0])

    @pl.when(g == _NCHUNK - 1)
    def _fin():
        out[:] = acc[:]


def _dist23_kernel(d1, f2, m2, f3, m3, out_idx, acc):
    g = pl.program_id(0)

    @pl.when(g == 0)
    def _init():
        acc[:] = jnp.zeros_like(acc)

    acc[:] += _partial_dist(f2[:], m2[:], _DS[1])
    acc[:] += _partial_dist(f3[:], m3[:], _DS[2])

    @pl.when(g == _NCHUNK - 1)
    def _fin():
        out_idx[0, :] = jnp.argmin(acc[:] + d1[:], axis=1).astype(jnp.int32)


def _compute_idx(f1, m1, f2, m2, f3, m3):
    c1 = _DS[0] // _NCHUNK
    d1 = pl.pallas_call(
        _dist1_kernel,
        grid=(_NCHUNK,),
        in_specs=[
            pl.BlockSpec((_B, c1), lambda i: (0, i)),
            pl.BlockSpec((_M, c1), lambda i: (0, i)),
        ],
        out_specs=pl.BlockSpec((_B, _M), lambda i: (0, 0)),
        out_shape=jax.ShapeDtypeStruct((_B, _M), jnp.float32),
        scratch_shapes=[pltpu.VMEM((_B, _M), jnp.float32)],
        compiler_params=pltpu.CompilerParams(
            dimension_semantics=("arbitrary",)
        ),
    )(f1, m1)

    c2 = _DS[1] // _NCHUNK
    c3 = _DS[2] // _NCHUNK
    idx = pl.pallas_call(
        _dist23_kernel,
        grid=(_NCHUNK,),
        in_specs=[
            pl.BlockSpec((_B, _M), lambda i: (0, 0)),
            pl.BlockSpec((_B, c2), lambda i: (0, i)),
            pl.BlockSpec((_M, c2), lambda i: (0, i)),
            pl.BlockSpec((_B, c3), lambda i: (0, i)),
            pl.BlockSpec((_M, c3), lambda i: (0, i)),
        ],
        out_specs=pl.BlockSpec((1, _B), lambda i: (0, 0)),
        out_shape=jax.ShapeDtypeStruct((1, _B), jnp.int32),
        scratch_shapes=[pltpu.VMEM((_B, _M), jnp.float32)],
        compiler_params=pltpu.CompilerParams(
            dimension_semantics=("arbitrary",)
        ),
    )(d1, f2, m2, f3, m3)
    return idx[0]


def _gather_kernel(idx_ref, f1, m1, f2, m2, f3, m3, o1, o2, o3):
    del idx_ref
    for f, m, o, (c, _, _) in (
        (f1, m1, o1, _SHAPES[0]),
        (f2, m2, o2, _SHAPES[1]),
        (f3, m3, o3, _SHAPES[2]),
    ):
        fv = f[0]
        mv = m[0]
        o[0, :c] = fv
        d = mv - fv
        o[0, c:] = d * d


def _compute_outputs(idx, f1, m1, f2, m2, f3, m3):
    in_specs = []
    out_specs = []
    out_shape = []
    for c, h, w in _SHAPES:
        in_specs.append(
            pl.BlockSpec((1, c, h * w), lambda b, idx_ref: (b, 0, 0))
        )
        in_specs.append(
            pl.BlockSpec((1, c, h * w), lambda b, idx_ref: (idx_ref[b], 0, 0))
        )
        out_specs.append(
            pl.BlockSpec((1, 2 * c, h * w), lambda b, idx_ref: (b, 0, 0))
        )
        out_shape.append(
            jax.ShapeDtypeStruct((_B, 2 * c, h * w), jnp.float32)
        )
    grid_spec = pltpu.PrefetchScalarGridSpec(
        num_scalar_prefetch=1,
        grid=(_B,),
        in_specs=in_specs,
        out_specs=out_specs,
    )
    return pl.pallas_call(
        _gather_kernel,
        grid_spec=grid_spec,
        out_shape=out_shape,
        compiler_params=pltpu.CompilerParams(
            dimension_semantics=("arbitrary",)
        ),
    )(idx, f1, m1, f2, m2, f3, m3)


@jax.jit
def kernel(feat1, feat2, feat3, mem1, mem2, mem3):
    feats = (feat1, feat2, feat3)
    mems = (mem1, mem2, mem3)
    ff = [f.reshape(_B, -1) for f in feats]
    mf = [m.reshape(_M, -1) for m in mems]
    idx = _compute_idx(ff[0], mf[0], ff[1], mf[1], ff[2], mf[2])

    f3d = [f.reshape(_B, c, h * w) for f, (c, h, w) in zip(feats, _SHAPES)]
    m3d = [m.reshape(_M, c, h * w) for m, (c, h, w) in zip(mems, _SHAPES)]
    outs = _compute_outputs(
        idx, f3d[0], m3d[0], f3d[1], m3d[1], f3d[2], m3d[2]
    )
    return tuple(
        o.reshape(_B, 2 * c, h, w) for o, (c, h, w) in zip(outs, _SHAPES)
    )


# speedup vs baseline: 1.2040x; 1.2040x over previous
import jax
import jax.numpy as jnp
from jax.experimental import pallas as pl
from jax.experimental.pallas import tpu as pltpu

_B = 32
_M = 30
_SHAPES = [(64, 64, 64), (128, 32, 32), (256, 16, 16)]
_DS = [c * h * w for (c, h, w) in _SHAPES]
_NCHUNK = 8
_L1CHUNK = 8


def _partial_dist(a, b, d):
    cross = jax.lax.dot_general(
        a, b, (((1,), (1,)), ((), ())), preferred_element_type=jnp.float32
    )
    a2 = jnp.sum(a * a, axis=1)
    b2 = jnp.sum(b * b, axis=1)
    return (a2[:, None] + b2[None, :] - 2.0 * cross) * (1.0 / d)


def _dist1_kernel(f1, m1, out, acc):
    g = pl.program_id(0)

    @pl.when(g == 0)
    def _init():
        acc[:] = jnp.zeros_like(acc)

    acc[:] += _partial_dist(f1[:], m1[:], _DS[0])

    @pl.when(g == _NCHUNK - 1)
    def _fin():
        out[:] = acc[:]


def _dist23_kernel(d1, f2, m2, f3, m3, out_idx, acc):
    g = pl.program_id(0)

    @pl.when(g == 0)
    def _init():
        acc[:] = jnp.zeros_like(acc)

    acc[:] += _partial_dist(f2[:], m2[:], _DS[1])
    acc[:] += _partial_dist(f3[:], m3[:], _DS[2])

    @pl.when(g == _NCHUNK - 1)
    def _fin():
        out_idx[0, :] = jnp.argmin(acc[:] + d1[:], axis=1).astype(jnp.int32)


def _compute_idx(ff1, mf1, ff2, mf2, ff3, mf3):
    c1 = _DS[0] // _NCHUNK
    d1 = pl.pallas_call(
        _dist1_kernel,
        grid=(_NCHUNK,),
        in_specs=[
            pl.BlockSpec((_B, c1), lambda i: (0, i)),
            pl.BlockSpec((_M, c1), lambda i: (0, i)),
        ],
        out_specs=pl.BlockSpec((_B, _M), lambda i: (0, 0)),
        out_shape=jax.ShapeDtypeStruct((_B, _M), jnp.float32),
        scratch_shapes=[pltpu.VMEM((_B, _M), jnp.float32)],
        compiler_params=pltpu.CompilerParams(
            dimension_semantics=("arbitrary",)
        ),
    )(ff1, mf1)

    c2 = _DS[1] // _NCHUNK
    c3 = _DS[2] // _NCHUNK
    idx = pl.pallas_call(
        _dist23_kernel,
        grid=(_NCHUNK,),
        in_specs=[
            pl.BlockSpec((_B, _M), lambda i: (0, 0)),
            pl.BlockSpec((_B, c2), lambda i: (0, i)),
            pl.BlockSpec((_M, c2), lambda i: (0, i)),
            pl.BlockSpec((_B, c3), lambda i: (0, i)),
            pl.BlockSpec((_M, c3), lambda i: (0, i)),
        ],
        out_specs=pl.BlockSpec((1, _B), lambda i: (0, 0)),
        out_shape=jax.ShapeDtypeStruct((1, _B), jnp.int32),
        scratch_shapes=[pltpu.VMEM((_B, _M), jnp.float32)],
        compiler_params=pltpu.CompilerParams(
            dimension_semantics=("arbitrary",)
        ),
    )(d1, ff2, mf2, ff3, mf3)
    return idx[0]


_L1GB = 8


def _l1_outputs(idx, ff1, mf1):
    d1 = _DS[0]
    ck = d1 // _L1CHUNK
    grid_spec = pltpu.PrefetchScalarGridSpec(
        num_scalar_prefetch=1,
        grid=(_B // _L1GB, _L1CHUNK),
        in_specs=[
            pl.BlockSpec((_L1GB, ck), lambda g, j, idx_ref: (g, j)),
            pl.BlockSpec(memory_space=pltpu.MemorySpace.HBM),
        ],
        out_specs=pl.BlockSpec(
            (_L1GB, 2, ck), lambda g, j, idx_ref: (g, 0, j)
        ),
        scratch_shapes=[
            pltpu.VMEM((_L1GB, ck), jnp.float32),
            pltpu.SemaphoreType.DMA((_L1GB,)),
        ],
    )

    def body(idx_ref, f, m_hbm, o, sel, sems):
        g = pl.program_id(0)
        j = pl.program_id(1)
        for i in range(_L1GB):
            row = idx_ref[g * _L1GB + i]
            pltpu.make_async_copy(
                m_hbm.at[row, pl.ds(j * ck, ck)], sel.at[i], sems.at[i]
            ).start()
        for i in range(_L1GB):
            row = idx_ref[g * _L1GB + i]
            pltpu.make_async_copy(
                m_hbm.at[row, pl.ds(j * ck, ck)], sel.at[i], sems.at[i]
            ).wait()
        fv = f[:]
        o[:, 0] = fv
        d = sel[:] - fv
        o[:, 1] = d * d

    of = pl.pallas_call(
        body,
        grid_spec=grid_spec,
        out_shape=jax.ShapeDtypeStruct((_B, 2, d1), jnp.float32),
        compiler_params=pltpu.CompilerParams(
            dimension_semantics=("arbitrary", "arbitrary")
        ),
    )(idx, ff1, mf1)
    c, h, w = _SHAPES[0]
    return of.reshape(_B, 2 * c, h, w)


def _l23_kernel(idx_ref, f2, m2, f3, m3, o2, o3):
    del idx_ref
    for f, m, o, (c, _, _) in (
        (f2, m2, o2, _SHAPES[1]),
        (f3, m3, o3, _SHAPES[2]),
    ):
        fv = f[0]
        mv = m[0]
        o[0, :, :c] = fv
        d = mv - fv
        o[0, :, c:] = d * d


def _l23_outputs(idx, fn2, mn2, fn3, mn3):
    in_specs = []
    out_specs = []
    out_shape = []
    for c, h, w in _SHAPES[1:]:
        in_specs.append(
            pl.BlockSpec((1, h * w, c), lambda b, idx_ref: (b, 0, 0))
        )
        in_specs.append(
            pl.BlockSpec(
                (1, h * w, c), lambda b, idx_ref: (idx_ref[b], 0, 0)
            )
        )
        out_specs.append(
            pl.BlockSpec((1, h * w, 2 * c), lambda b, idx_ref: (b, 0, 0))
        )
        out_shape.append(
            jax.ShapeDtypeStruct((_B, h * w, 2 * c), jnp.float32)
        )
    grid_spec = pltpu.PrefetchScalarGridSpec(
        num_scalar_prefetch=1,
        grid=(_B,),
        in_specs=in_specs,
        out_specs=out_specs,
    )
    on2, on3 = pl.pallas_call(
        _l23_kernel,
        grid_spec=grid_spec,
        out_shape=out_shape,
        compiler_params=pltpu.CompilerParams(
            dimension_semantics=("arbitrary",)
        ),
    )(idx, fn2, mn2, fn3, mn3)
    outs = []
    for on, (c, h, w) in zip((on2, on3), _SHAPES[1:]):
        outs.append(jnp.transpose(on, (0, 2, 1)).reshape(_B, 2 * c, h, w))
    return outs


@jax.jit
def kernel(feat1, feat2, feat3, mem1, mem2, mem3):
    ff1 = feat1.reshape(_B, -1)
    mf1 = mem1.reshape(_M, -1)
    fn2 = jnp.transpose(feat2, (0, 2, 3, 1)).reshape(_B, _SHAPES[1][1] * _SHAPES[1][2], _SHAPES[1][0])
    mn2 = jnp.transpose(mem2, (0, 2, 3, 1)).reshape(_M, _SHAPES[1][1] * _SHAPES[1][2], _SHAPES[1][0])
    fn3 = jnp.transpose(feat3, (0, 2, 3, 1)).reshape(_B, _SHAPES[2][1] * _SHAPES[2][2], _SHAPES[2][0])
    mn3 = jnp.transpose(mem3, (0, 2, 3, 1)).reshape(_M, _SHAPES[2][1] * _SHAPES[2][2], _SHAPES[2][0])
    ff2 = fn2.reshape(_B, -1)
    mf2 = mn2.reshape(_M, -1)
    ff3 = fn3.reshape(_B, -1)
    mf3 = mn3.reshape(_M, -1)

    idx = _compute_idx(ff1, mf1, ff2, mf2, ff3, mf3)

    o1 = _l1_outputs(idx, ff1, mf1)
    o2, o3 = _l23_outputs(idx, fn2, mn2, fn3, mn3)
    return (o1, o2, o3)


# speedup vs baseline: 2.0339x; 1.6894x over previous
import jax
import jax.numpy as jnp
from jax.experimental import pallas as pl
from jax.experimental.pallas import tpu as pltpu

_B = 32
_M = 30
_SHAPES = [(64, 64, 64), (128, 32, 32), (256, 16, 16)]
_DS = [c * h * w for (c, h, w) in _SHAPES]
_NCHUNK = 8


def _partial_dist(a, b, d):
    cross = jax.lax.dot_general(
        a, b, (((1,), (1,)), ((), ())), preferred_element_type=jnp.float32
    )
    a2 = jnp.sum(a * a, axis=1)
    b2 = jnp.sum(b * b, axis=1)
    return (a2[:, None] + b2[None, :] - 2.0 * cross) * (1.0 / d)


def _dist1_kernel(f1, m1, out, acc):
    g = pl.program_id(0)

    @pl.when(g == 0)
    def _init():
        acc[:] = jnp.zeros_like(acc)

    acc[:] += _partial_dist(f1[:], m1[:], _DS[0])

    @pl.when(g == _NCHUNK - 1)
    def _fin():
        out[:] = acc[:]


def _dist23_kernel(d1, f2, m2, f3, m3, out_idx, acc):
    g = pl.program_id(0)

    @pl.when(g == 0)
    def _init():
        acc[:] = jnp.zeros_like(acc)

    acc[:] += _partial_dist(f2[:], m2[:], _DS[1])
    acc[:] += _partial_dist(f3[:], m3[:], _DS[2])

    @pl.when(g == _NCHUNK - 1)
    def _fin():
        out_idx[0, :] = jnp.argmin(acc[:] + d1[:], axis=1).astype(jnp.int32)


def _compute_idx(ff1, mf1, ff2, mf2, ff3, mf3):
    c1 = _DS[0] // _NCHUNK
    d1 = pl.pallas_call(
        _dist1_kernel,
        grid=(_NCHUNK,),
        in_specs=[
            pl.BlockSpec((_B, c1), lambda i: (0, i)),
            pl.BlockSpec((_M, c1), lambda i: (0, i)),
        ],
        out_specs=pl.BlockSpec((_B, _M), lambda i: (0, 0)),
        out_shape=jax.ShapeDtypeStruct((_B, _M), jnp.float32),
        scratch_shapes=[pltpu.VMEM((_B, _M), jnp.float32)],
        compiler_params=pltpu.CompilerParams(
            dimension_semantics=("arbitrary",)
        ),
    )(ff1, mf1)

    c2 = _DS[1] // _NCHUNK
    c3 = _DS[2] // _NCHUNK
    idx = pl.pallas_call(
        _dist23_kernel,
        grid=(_NCHUNK,),
        in_specs=[
            pl.BlockSpec((_B, _M), lambda i: (0, 0)),
            pl.BlockSpec((_B, c2), lambda i: (0, i)),
            pl.BlockSpec((_M, c2), lambda i: (0, i)),
            pl.BlockSpec((_B, c3), lambda i: (0, i)),
            pl.BlockSpec((_M, c3), lambda i: (0, i)),
        ],
        out_specs=pl.BlockSpec((1, _B), lambda i: (0, 0)),
        out_shape=jax.ShapeDtypeStruct((1, _B), jnp.int32),
        scratch_shapes=[pltpu.VMEM((_B, _M), jnp.float32)],
        compiler_params=pltpu.CompilerParams(
            dimension_semantics=("arbitrary",)
        ),
    )(d1, ff2, mf2, ff3, mf3)
    return idx[0]


def _gather_kernel(idx_ref, f1, m1, f2, m2, f3, m3, o1, o2, o3):
    del idx_ref
    for f, m, o, (c, _, _) in (
        (f1, m1, o1, _SHAPES[0]),
        (f2, m2, o2, _SHAPES[1]),
        (f3, m3, o3, _SHAPES[2]),
    ):
        fv = f[0]
        mv = m[0]
        o[0, :, :c] = fv
        d = mv - fv
        o[0, :, c:] = d * d


def _compute_outputs(idx, fn1, mn1, fn2, mn2, fn3, mn3):
    in_specs = []
    out_specs = []
    out_shape = []
    for c, h, w in _SHAPES:
        in_specs.append(
            pl.BlockSpec((1, h * w, c), lambda b, idx_ref: (b, 0, 0))
        )
        in_specs.append(
            pl.BlockSpec(
                (1, h * w, c), lambda b, idx_ref: (idx_ref[b], 0, 0)
            )
        )
        out_specs.append(
            pl.BlockSpec((1, h * w, 2 * c), lambda b, idx_ref: (b, 0, 0))
        )
        out_shape.append(
            jax.ShapeDtypeStruct((_B, h * w, 2 * c), jnp.float32)
        )
    grid_spec = pltpu.PrefetchScalarGridSpec(
        num_scalar_prefetch=1,
        grid=(_B,),
        in_specs=in_specs,
        out_specs=out_specs,
    )
    ons = pl.pallas_call(
        _gather_kernel,
        grid_spec=grid_spec,
        out_shape=out_shape,
        compiler_params=pltpu.CompilerParams(
            dimension_semantics=("arbitrary",)
        ),
    )(idx, fn1, mn1, fn2, mn2, fn3, mn3)
    outs = []
    for on, (c, h, w) in zip(ons, _SHAPES):
        outs.append(jnp.transpose(on, (0, 2, 1)).reshape(_B, 2 * c, h, w))
    return outs


def _nhwc(x, c, h, w):
    return jnp.transpose(x, (0, 2, 3, 1)).reshape(x.shape[0], h * w, c)


@jax.jit
def kernel(feat1, feat2, feat3, mem1, mem2, mem3):
    fn1 = _nhwc(feat1, *_SHAPES[0])
    mn1 = _nhwc(mem1, *_SHAPES[0])
    fn2 = _nhwc(feat2, *_SHAPES[1])
    mn2 = _nhwc(mem2, *_SHAPES[1])
    fn3 = _nhwc(feat3, *_SHAPES[2])
    mn3 = _nhwc(mem3, *_SHAPES[2])

    ff1 = feat1.reshape(_B, -1)
    mf1 = mem1.reshape(_M, -1)
    ff2 = fn2.reshape(_B, -1)
    mf2 = mn2.reshape(_M, -1)
    ff3 = fn3.reshape(_B, -1)
    mf3 = mn3.reshape(_M, -1)

    idx = _compute_idx(ff1, mf1, ff2, mf2, ff3, mf3)
    return tuple(_compute_outputs(idx, fn1, mn1, fn2, mn2, fn3, mn3))
